# baseline (speedup 1.0000x reference)
import functools

import jax
import jax.numpy as jnp
from jax import lax
from jax.experimental import pallas as pl
from jax.experimental.pallas import tpu as pltpu
from jax.experimental.pallas import tpu_sc as plsc

_NC = 2
_NS = 16
_L = 16


def _rsqrt_newton(v):
    i = plsc.bitcast(v, jnp.int32)
    i = jnp.int32(0x5F3759DF) - lax.shift_right_logical(i, 1)
    y = plsc.bitcast(i, jnp.float32)
    for _ in range(3):
        y = y * (1.5 - 0.5 * v * y * y)
    return y


def _make_sc_kernel(B, V, D):
    nw = _NC * _NS
    b_per_w = B // nw
    n_groups = b_per_w // _L
    mesh = plsc.VectorSubcoreMesh(core_axis_name="c", subcore_axis_name="s")

    @functools.partial(
        pl.kernel,
        out_type=jax.ShapeDtypeStruct((B, D), jnp.float32),
        mesh=mesh,
        scratch_types=[
            pltpu.VMEM((b_per_w,), jnp.int32),
            pltpu.VMEM((b_per_w, D), jnp.float32),
            pltpu.VMEM((b_per_w, D), jnp.float32),
            pltpu.VMEM((D,), jnp.float32),
            pltpu.VMEM((D,), jnp.float32),
            pltpu.SemaphoreType.DMA,
        ],
        compiler_params=pltpu.CompilerParams(
            needs_layout_passes=False, use_tc_tiling_on_sc=False
        ),
    )
    def sc_kernel(x_hbm, table_hbm, gamma_hbm, beta_hbm, out_hbm,
                  idx_v, rows_v, out_v, g_v, b_v, sem):
        wid = lax.axis_index("s") * _NC + lax.axis_index("c")
        base = wid * b_per_w
        pltpu.sync_copy(x_hbm.at[pl.ds(base, b_per_w)], idx_v)
        pltpu.sync_copy(gamma_hbm, g_v)
        pltpu.sync_copy(beta_hbm, b_v)
        pltpu.async_copy(table_hbm.at[idx_v], rows_v, sem).wait()

        lane = lax.iota(jnp.int32, _L)
        inv_d = jnp.float32(1.0 / D)

        def group_body(g, carry):
            rows = g * _L + lane
            s = jnp.zeros((_L,), jnp.float32)
            ss = jnp.zeros((_L,), jnp.float32)
            for c in range(D):
                col = jnp.full((_L,), c, jnp.int32)
                v = plsc.load_gather(rows_v, [rows, col])
                s = s + v
                ss = ss + v * v
            mean = s * inv_d
            var = ss * inv_d - mean * mean
            rstd = _rsqrt_newton(var + jnp.float32(1e-5))
            for c in range(D):
                col = jnp.full((_L,), c, jnp.int32)
                v = plsc.load_gather(rows_v, [rows, col])
                gc = plsc.load_gather(g_v, [col])
                bc = plsc.load_gather(b_v, [col])
                o = (v - mean) * rstd * gc + bc
                plsc.store_scatter(out_v, [rows, col], o)
            return carry

        lax.fori_loop(0, n_groups, group_body, jnp.int32(0))
        pltpu.sync_copy(out_v, out_hbm.at[pl.ds(base, b_per_w)])

    return sc_kernel


def kernel(x, table, gamma, beta):
    B = x.shape[0]
    V, D = table.shape
    sc = _make_sc_kernel(B, V, D)
    return sc(x.astype(jnp.int32), table, gamma, beta)


# speedup vs baseline: 1.4349x; 1.4349x over previous
import functools

import jax
import jax.numpy as jnp
from jax import lax
from jax.experimental import pallas as pl
from jax.experimental.pallas import tpu as pltpu
from jax.experimental.pallas import tpu_sc as plsc

_NC = 2
_NS = 16
_L = 16


def _rsqrt_newton(v):
    i = plsc.bitcast(v, jnp.int32)
    i = jnp.int32(0x5F3759DF) - lax.shift_right_logical(i, 1)
    y = plsc.bitcast(i, jnp.float32)
    for _ in range(3):
        y = y * (1.5 - 0.5 * v * y * y)
    return y


def _make_sc_kernel(B, V, D):
    nw = _NC * _NS
    b_per_w = B // nw
    n_groups = b_per_w // _L
    d2 = 2 * D
    mesh = plsc.VectorSubcoreMesh(core_axis_name="c", subcore_axis_name="s")

    @functools.partial(
        pl.kernel,
        out_type=jax.ShapeDtypeStruct((D, B), jnp.float32),
        mesh=mesh,
        scratch_types=[
            pltpu.VMEM((b_per_w,), jnp.int32),
            pltpu.VMEM((b_per_w,), jnp.int32),
            pltpu.VMEM((b_per_w, d2), jnp.float32),
            pltpu.VMEM((D, b_per_w), jnp.float32),
            pltpu.VMEM((D,), jnp.float32),
            pltpu.VMEM((D,), jnp.float32),
            pltpu.SemaphoreType.DMA,
        ],
        compiler_params=pltpu.CompilerParams(needs_layout_passes=False),
    )
    def sc_kernel(x_hbm, table_hbm, gamma_hbm, beta_hbm, out_hbm,
                  x_v, idx2_v, rows_v, out_t, g_v, b_v, sem):
        wid = lax.axis_index("s") * _NC + lax.axis_index("c")
        base = wid * b_per_w
        pltpu.sync_copy(x_hbm.at[pl.ds(base, b_per_w)], x_v)
        pltpu.sync_copy(gamma_hbm, g_v)
        pltpu.sync_copy(beta_hbm, b_v)

        lane = lax.iota(jnp.int32, _L)

        def idx_body(i, carry):
            xg = x_v[pl.ds(i * _L, _L)]
            idx2_v[pl.ds(i * _L, _L)] = lax.shift_right_logical(xg, 1)
            return carry

        lax.fori_loop(0, n_groups, idx_body, jnp.int32(0))

        pltpu.async_copy(table_hbm.at[idx2_v], rows_v, sem).wait()

        inv_d = jnp.float32(1.0 / D)

        def group_body(g, carry):
            rows = g * _L + lane
            xg = x_v[pl.ds(g * _L, _L)]
            par = lax.shift_left(jnp.bitwise_and(xg, 1), 6)
            s = jnp.zeros((_L,), jnp.float32)
            ss = jnp.zeros((_L,), jnp.float32)
            for c in range(D):
                q = c & ~15
                rot = jnp.bitwise_and(lane + c, 15)
                col = q + rot
                v = plsc.load_gather(rows_v, [rows, par + col])
                s = s + v
                ss = ss + v * v
            mean = s * inv_d
            var = ss * inv_d - mean * mean
            rstd = _rsqrt_newton(var + jnp.float32(1e-5))
            for c in range(D):
                q = c & ~15
                rot = jnp.bitwise_and(lane + c, 15)
                col = q + rot
                v = plsc.load_gather(rows_v, [rows, par + col])
                gc = plsc.load_gather(g_v, [col])
                bc = plsc.load_gather(b_v, [col])
                o = (v - mean) * rstd * gc + bc
                plsc.store_scatter(out_t, [col, rows], o)
            return carry

        lax.fori_loop(0, n_groups, group_body, jnp.int32(0))
        pltpu.sync_copy(out_t, out_hbm.at[:, pl.ds(base, b_per_w)])

    return sc_kernel


def kernel(x, table, gamma, beta):
    B = x.shape[0]
    V, D = table.shape
    sc = _make_sc_kernel(B, V, D)
    table_pairs = table.reshape(V // 2, 2 * D)
    out_t = sc(x.astype(jnp.int32), table_pairs, gamma, beta)
    return out_t.T


# speedup vs baseline: 2.6130x; 1.8210x over previous
import functools

import jax
import jax.numpy as jnp
from jax import lax
from jax.experimental import pallas as pl
from jax.experimental.pallas import tpu as pltpu
from jax.experimental.pallas import tpu_sc as plsc

_NC = 2
_NS = 16
_L = 16
_XCHUNK = 4096


def _make_gather_kernel(B, V, D):
    nw = _NC * _NS
    d_per_w = D // nw
    n_chunks = B // _XCHUNK
    mesh = plsc.VectorSubcoreMesh(core_axis_name="c", subcore_axis_name="s")

    @functools.partial(
        pl.kernel,
        out_type=jax.ShapeDtypeStruct((D, B), jnp.float32),
        mesh=mesh,
        scratch_types=[
            pltpu.VMEM((V,), jnp.float32),
            pltpu.VMEM((_XCHUNK,), jnp.int32),
            pltpu.VMEM((_XCHUNK,), jnp.float32),
        ],
        compiler_params=pltpu.CompilerParams(needs_layout_passes=False),
    )
    def gather_kernel(x_hbm, t_hbm, g_hbm, row_v, x_v, o_v):
        wid = lax.axis_index("s") * _NC + lax.axis_index("c")

        def per_dim(d):
            pltpu.sync_copy(t_hbm.at[d], row_v)
            for chunk in range(n_chunks):
                pltpu.sync_copy(x_hbm.at[pl.ds(chunk * _XCHUNK, _XCHUNK)], x_v)

                def body(j, carry):
                    idx = x_v[pl.ds(j * _L, _L)]
                    o_v[pl.ds(j * _L, _L)] = plsc.load_gather(row_v, [idx])
                    return carry

                lax.fori_loop(0, _XCHUNK // _L, body, jnp.int32(0))
                pltpu.sync_copy(o_v, g_hbm.at[d, pl.ds(chunk * _XCHUNK, _XCHUNK)])

        for k in range(d_per_w):
            per_dim(wid * d_per_w + k)

    return gather_kernel


def _ln_block(g_ref, gamma_ref, beta_ref, o_ref):
    g = g_ref[...]
    d = g.shape[0]
    mean = jnp.mean(g, axis=0, keepdims=True)
    var = jnp.mean(g * g, axis=0, keepdims=True) - mean * mean
    rstd = lax.rsqrt(var + jnp.float32(1e-5))
    o_ref[...] = (g - mean) * rstd * gamma_ref[...] + beta_ref[...]


def _make_ln_kernel(B, D, blk=2048):
    grid = (B // blk,)
    return pl.pallas_call(
        _ln_block,
        grid=grid,
        in_specs=[
            pl.BlockSpec((D, blk), lambda i: (0, i)),
            pl.BlockSpec((D, 1), lambda i: (0, 0)),
            pl.BlockSpec((D, 1), lambda i: (0, 0)),
        ],
        out_specs=pl.BlockSpec((D, blk), lambda i: (0, i)),
        out_shape=jax.ShapeDtypeStruct((D, B), jnp.float32),
    )


def kernel(x, table, gamma, beta):
    B = x.shape[0]
    V, D = table.shape
    gathered = _make_gather_kernel(B, V, D)(x.astype(jnp.int32), table.T)
    out_t = _make_ln_kernel(B, D)(
        gathered, gamma.reshape(D, 1), beta.reshape(D, 1)
    )
    return out_t.T
